# baseline (speedup 1.0000x reference)
import functools
import jax
import jax.numpy as jnp
from jax import lax
from jax.experimental import pallas as pl
from jax.experimental.pallas import tpu as pltpu
from jax.experimental.pallas import tpu_sc as plsc

TOPK = 8
L = 16
RCHUNK = 8
UNROLL = 32


def _bfly(v, iota, op):
    for s in (8, 4, 2, 1):
        perm = jnp.bitwise_xor(iota, s)
        v = op(v, v.at[perm].get(mode="promise_in_bounds"))
    return v


def _make_sc_router(B, C, S, E):
    NCH = C // RCHUNK
    NQ = E // L
    NJ = (S // L) // UNROLL
    inv_s = 1.0 / S

    mesh = plsc.VectorSubcoreMesh(core_axis_name="c", subcore_axis_name="s")

    @functools.partial(
        pl.kernel,
        mesh=mesh,
        out_type=[
            jax.ShapeDtypeStruct((B, L), jnp.float32),
            jax.ShapeDtypeStruct((B, L), jnp.int32),
        ],
        scratch_types=[
            pltpu.VMEM((2, RCHUNK, S), jnp.float32),
            pltpu.VMEM((C,), jnp.float32),
            pltpu.VMEM((C, E), jnp.float32),
            pltpu.VMEM((E,), jnp.float32),
            pltpu.VMEM((E,), jnp.float32),
            pltpu.VMEM((L,), jnp.float32),
            pltpu.VMEM((L,), jnp.int32),
            pltpu.SemaphoreType.DMA((2,)),
        ],
    )
    def sc_router(x_hbm, gwt_hbm, gb_hbm, eb_hbm, probs_hbm, idx_hbm,
                  buf, sums_v, gwv, gbv, ebv, pbuf, ibuf, sems):
        w = lax.axis_index("s") * 2 + lax.axis_index("c")
        iota = lax.broadcasted_iota(jnp.int32, (L,), 0)

        def cp(g, slot):
            return pltpu.make_async_copy(
                x_hbm.at[w, pl.ds(g * RCHUNK, RCHUNK), :],
                buf.at[slot], sems.at[slot])

        cp(0, 0).start()
        cp(1, 1).start()
        pltpu.sync_copy(gwt_hbm, gwv)
        pltpu.sync_copy(gb_hbm, gbv)
        pltpu.sync_copy(eb_hbm, ebv)

        def chunk_body(g, rv):
            slot = lax.rem(g, 2)
            cp(g, slot).wait()
            for r in range(RCHUNK):
                def jbody(j, accs):
                    base = j * (UNROLL * L)
                    out = list(accs)
                    for u in range(UNROLL):
                        v = buf[slot, r, pl.ds(base + u * L, L)]
                        out[u % 4] = out[u % 4] + v
                    return tuple(out)
                z = jnp.zeros((L,), jnp.float32)
                a0, a1, a2, a3 = lax.fori_loop(0, NJ, jbody, (z, z, z, z))
                acc = _bfly((a0 + a1) + (a2 + a3), iota, jnp.add)
                rv = jnp.where(iota == slot * RCHUNK + r, acc * inv_s, rv)

            @pl.when(lax.rem(g, 2) == 1)
            def _():
                sums_v[pl.ds((g // 2) * L, L)] = rv

            @pl.when(g + 2 < NCH)
            def _():
                cp(g + 2, lax.rem(g + 2, 2)).start()
            return rv

        lax.fori_loop(0, NCH, chunk_body, jnp.zeros((L,), jnp.float32))

        def mac_body(t, accs):
            sv = sums_v[pl.ds(t * L, L)]
            out = list(accs)
            for u in range(L):
                lane = jnp.full((L,), u, jnp.int32)
                bc = sv.at[lane].get(mode="promise_in_bounds")
                for q in range(NQ):
                    out[q] = out[q] + bc * gwv[t * L + u, pl.ds(q * L, L)]
            return tuple(out)
        z = jnp.zeros((L,), jnp.float32)
        lg = list(lax.fori_loop(0, C // L, mac_body,
                                tuple(z for _ in range(NQ))))

        for q in range(NQ):
            lq = lg[q] + gbv[pl.ds(q * L, L)]
            lq = jnp.clip(lq, -10.0, 10.0)
            lg[q] = lq + ebv[pl.ds(q * L, L)]

        mv = jnp.maximum(jnp.maximum(lg[0], lg[1]),
                         jnp.maximum(lg[2], lg[3]))
        mxv = _bfly(mv, iota, jnp.maximum)
        eq = [jnp.exp(lg[q] - mxv) for q in range(NQ)]
        esv = _bfly((eq[0] + eq[1]) + (eq[2] + eq[3]), iota, jnp.add)
        p = [jnp.clip(e / esv, 1e-06, 1.0) for e in eq]

        topv = jnp.zeros((L,), jnp.float32)
        topi = jnp.zeros((L,), jnp.int32)
        psum = jnp.zeros((L,), jnp.float32)
        sel = list(lg)
        for k in range(TOPK):
            m = jnp.maximum(jnp.maximum(sel[0], sel[1]),
                            jnp.maximum(sel[2], sel[3]))
            mkv = _bfly(m, iota, jnp.maximum)
            cand = [jnp.where(sel[q] == mkv, iota + q * L, E)
                    for q in range(NQ)]
            cm = jnp.minimum(jnp.minimum(cand[0], cand[1]),
                             jnp.minimum(cand[2], cand[3]))
            civ = _bfly(cm, iota, jnp.minimum)
            hit = [iota + q * L == civ for q in range(NQ)]
            pv = jnp.where(hit[0], p[0], 0.0)
            for q in range(1, NQ):
                pv = pv + jnp.where(hit[q], p[q], 0.0)
            pkv = _bfly(pv, iota, jnp.add)
            topv = jnp.where(iota == k, pkv, topv)
            topi = jnp.where(iota == k, civ, topi)
            psum = psum + pkv
            for q in range(NQ):
                sel[q] = jnp.where(hit[q], -3.0e38, sel[q])

        pbuf[...] = topv / (psum + 1e-08)
        ibuf[...] = topi
        pltpu.sync_copy(pbuf, probs_hbm.at[w])
        pltpu.sync_copy(ibuf, idx_hbm.at[w])

    return sc_router


def kernel(x, gate_w, gate_b, expert_bias):
    B, C, H, W = x.shape
    E = gate_w.shape[0]
    S = H * W
    xr = x.reshape(B, C, S)
    gwt = gate_w.T.copy()

    probs16, idx16 = _make_sc_router(B, C, S, E)(xr, gwt, gate_b, expert_bias)
    probs = probs16[:, :TOPK]
    idx = idx16[:, :TOPK]
    loss = jnp.zeros((), dtype=jnp.float32)
    return (probs, idx, loss)
